# baseline (speedup 1.0000x reference)
import functools

import jax
import jax.numpy as jnp
from jax.experimental import pallas as pl
from jax.experimental.pallas import tpu as pltpu

_PREC = jax.lax.Precision.DEFAULT
_LANES = 128


def _main_body(e_num, s_len, fblk, alpha_ref, x_ref, w1_ref, b1_ref, w2_ref,
               b2_ref, wr_ref, out_ref, idx_ref, gate_ref, a_ref):
    fb = pl.program_id(1)

    @pl.when(fb == 0)
    def _route():
        pooled = jnp.sum(x_ref[0], axis=0, keepdims=True) * (1.0 / s_len)
        logits = jnp.dot(pooled, wr_ref[...], precision=_PREC,
                         preferred_element_type=jnp.float32)
        lane = jax.lax.broadcasted_iota(jnp.int32, logits.shape, 1)
        valid = lane < e_num
        l = jnp.where(valid, logits, jnp.float32(-1e30))
        p = jnp.exp(l - jnp.max(l))
        p = jnp.where(valid, p, 0.0)
        p = p / jnp.sum(p)
        big = jnp.int32(1 << 20)
        v1 = jnp.max(p)
        i1 = jnp.min(jnp.where(p >= v1, lane, big))
        p2 = jnp.where(lane == i1, jnp.float32(-1.0), p)
        v2 = jnp.max(p2)
        i2 = jnp.min(jnp.where(p2 >= v2, lane, big))
        a = jnp.clip(alpha_ref[0], 0.1, 1.0)
        scale = (1.0 - a) / (v1 + v2)
        gate_ref[0] = jnp.where(lane == 0, scale * v1,
                                jnp.where(lane == 1, scale * v2, 0.0))
        idx_ref[0] = jnp.where(lane == 0, i1,
                               jnp.where(lane == 1, i2, 0)).astype(jnp.int32)
        a_ref[0] = a

    a = a_ref[0]
    w1 = w1_ref[:, pl.ds(fb * fblk, fblk)]
    h = jnp.dot(x_ref[0], w1, precision=_PREC,
                preferred_element_type=jnp.float32)
    h = jax.nn.gelu(h + b1_ref[0, pl.ds(fb * fblk, fblk)])
    w2 = w2_ref[pl.ds(fb * fblk, fblk), :]
    y = jnp.dot(h, w2, precision=_PREC, preferred_element_type=jnp.float32)

    @pl.when(fb == 0)
    def _init():
        out_ref[0] = a * (y + b2_ref[0])

    @pl.when(fb != 0)
    def _acc():
        out_ref[0] += a * y


def _expert_body(eidx_ref, egate_ref, x_ref, main_ref, w1_ref, b1_ref,
                 w2_ref, b2_ref, out_ref):
    b = pl.program_id(0)
    fb = pl.program_id(1)
    br = pl.program_id(2)
    g = egate_ref[b, br]
    h = jnp.dot(x_ref[0], w1_ref[0], precision=_PREC,
                preferred_element_type=jnp.float32)
    h = jax.nn.gelu(h + b1_ref[0, 0])
    y = jnp.dot(h, w2_ref[0], precision=_PREC,
                preferred_element_type=jnp.float32)

    first = jnp.logical_and(fb == 0, br == 0)

    @pl.when(first)
    def _init():
        out_ref[0] = main_ref[0] + g * (y + b2_ref[0, 0])

    @pl.when(jnp.logical_and(fb == 0, br != 0))
    def _acc_bias():
        out_ref[0] += g * (y + b2_ref[0, 0])

    @pl.when(fb != 0)
    def _acc():
        out_ref[0] += g * y


def kernel(x, W1, b1, W2, b2, Wr, eW1, eb1, eW2, eb2, alpha):
    B, S, D = x.shape
    F = W1.shape[1]
    E = eW1.shape[0]
    K = 2

    FBLK = 512 if F % 512 == 0 else F
    NFB = F // FBLK

    wr_pad = jnp.zeros((D, _LANES), jnp.float32).at[:, :E].set(Wr)
    alpha_arr = jnp.reshape(alpha.astype(jnp.float32), (1,))

    main_out, idx_pad, gate_pad = pl.pallas_call(
        functools.partial(_main_body, E, S, FBLK),
        grid=(B, NFB),
        in_specs=[
            pl.BlockSpec(memory_space=pltpu.SMEM),
            pl.BlockSpec((1, S, D), lambda b, fb: (b, 0, 0)),
            pl.BlockSpec((D, F), lambda b, fb: (0, 0)),
            pl.BlockSpec((1, F), lambda b, fb: (0, 0)),
            pl.BlockSpec((F, D), lambda b, fb: (0, 0)),
            pl.BlockSpec((1, D), lambda b, fb: (0, 0)),
            pl.BlockSpec((D, _LANES), lambda b, fb: (0, 0)),
        ],
        out_specs=[
            pl.BlockSpec((1, S, D), lambda b, fb: (b, 0, 0)),
            pl.BlockSpec((1, 1, _LANES), lambda b, fb: (b, 0, 0)),
            pl.BlockSpec((1, 1, _LANES), lambda b, fb: (b, 0, 0)),
        ],
        out_shape=[
            jax.ShapeDtypeStruct((B, S, D), jnp.float32),
            jax.ShapeDtypeStruct((B, 1, _LANES), jnp.int32),
            jax.ShapeDtypeStruct((B, 1, _LANES), jnp.float32),
        ],
        scratch_shapes=[pltpu.SMEM((1,), jnp.float32)],
        compiler_params=pltpu.CompilerParams(
            dimension_semantics=("arbitrary", "arbitrary"),
            vmem_limit_bytes=100 * 1024 * 1024,
        ),
    )(alpha_arr, x, W1, b1[None, :], W2, b2[None, :], wr_pad)

    eidx = idx_pad[:, 0, :K]
    egate = gate_pad[:, 0, :K]

    EFBLK = 256 if F % 256 == 0 else F
    ENFB = F // EFBLK

    grid_spec = pltpu.PrefetchScalarGridSpec(
        num_scalar_prefetch=2,
        grid=(B, ENFB, K),
        in_specs=[
            pl.BlockSpec((1, S, D), lambda b, fb, br, eidx, egate: (b, 0, 0)),
            pl.BlockSpec((1, S, D), lambda b, fb, br, eidx, egate: (b, 0, 0)),
            pl.BlockSpec((1, D, EFBLK),
                         lambda b, fb, br, eidx, egate: (eidx[b, br], 0, fb)),
            pl.BlockSpec((1, 1, EFBLK),
                         lambda b, fb, br, eidx, egate: (eidx[b, br], 0, fb)),
            pl.BlockSpec((1, EFBLK, D),
                         lambda b, fb, br, eidx, egate: (eidx[b, br], fb, 0)),
            pl.BlockSpec((1, 1, D),
                         lambda b, fb, br, eidx, egate: (eidx[b, br], 0, 0)),
        ],
        out_specs=pl.BlockSpec((1, S, D),
                               lambda b, fb, br, eidx, egate: (b, 0, 0)),
    )
    out = pl.pallas_call(
        _expert_body,
        grid_spec=grid_spec,
        out_shape=jax.ShapeDtypeStruct((B, S, D), jnp.float32),
        compiler_params=pltpu.CompilerParams(
            dimension_semantics=("arbitrary", "arbitrary", "arbitrary"),
            vmem_limit_bytes=100 * 1024 * 1024,
        ),
    )(eidx, egate, x, main_out, eW1, eb1[:, None, :], eW2, eb2[:, None, :])
    return out


# speedup vs baseline: 1.4893x; 1.4893x over previous
import functools

import jax
import jax.numpy as jnp
from jax.experimental import pallas as pl
from jax.experimental.pallas import tpu as pltpu

_PREC = jax.lax.Precision.DEFAULT
_LANES = 128


def _router_body(e_num, s_len, alpha_ref, x_ref, wr_ref, idx_ref, gate_ref):
    pooled = jnp.sum(x_ref[0], axis=0, keepdims=True) * (1.0 / s_len)
    logits = jnp.dot(pooled, wr_ref[...], precision=_PREC,
                     preferred_element_type=jnp.float32)
    lane = jax.lax.broadcasted_iota(jnp.int32, logits.shape, 1)
    valid = lane < e_num
    l = jnp.where(valid, logits, jnp.float32(-1e30))
    p = jnp.exp(l - jnp.max(l))
    p = jnp.where(valid, p, 0.0)
    p = p / jnp.sum(p)
    big = jnp.int32(1 << 20)
    v1 = jnp.max(p)
    i1 = jnp.min(jnp.where(p >= v1, lane, big))
    p2 = jnp.where(lane == i1, jnp.float32(-1.0), p)
    v2 = jnp.max(p2)
    i2 = jnp.min(jnp.where(p2 >= v2, lane, big))
    a = jnp.clip(alpha_ref[0], 0.1, 1.0)
    scale = (1.0 - a) / (v1 + v2)
    gate_ref[0] = jnp.where(lane == 0, a,
                            jnp.where(lane == 1, scale * v1,
                                      jnp.where(lane == 2, scale * v2, 0.0)))
    idx_row = jnp.where(lane == 2, i2, i1)
    idx_ref[0] = idx_row.astype(jnp.int32)


def _mlp_branch(x_ref, w1, b1, w2, b2, g, fb, out_ref):
    h = jnp.dot(x_ref[0], w1, precision=_PREC,
                preferred_element_type=jnp.float32)
    h = jax.nn.gelu(h + b1)
    y = jnp.dot(h, w2, precision=_PREC, preferred_element_type=jnp.float32)
    br = pl.program_id(2)

    @pl.when(jnp.logical_and(fb == 0, br == 0))
    def _init():
        out_ref[0] = g * (y + b2)

    @pl.when(jnp.logical_and(fb == 0, br != 0))
    def _acc_bias():
        out_ref[0] += g * (y + b2)

    @pl.when(fb != 0)
    def _acc():
        out_ref[0] += g * y


def _dispatch_body(bidx_ref, gate_ref, x_ref, mw1_ref, mb1_ref, mw2_ref,
                   mb2_ref, ew1_ref, eb1_ref, ew2_ref, eb2_ref, out_ref):
    b = pl.program_id(0)
    fb = pl.program_id(1)
    br = pl.program_id(2)
    g = gate_ref[b, br]

    @pl.when(br == 0)
    def _main():
        _mlp_branch(x_ref, mw1_ref[...], mb1_ref[0], mw2_ref[...],
                    mb2_ref[0], g, fb, out_ref)

    @pl.when(br != 0)
    def _expert():
        _mlp_branch(x_ref, ew1_ref[0], eb1_ref[0, 0], ew2_ref[0],
                    eb2_ref[0, 0], g, fb, out_ref)


def kernel(x, W1, b1, W2, b2, Wr, eW1, eb1, eW2, eb2, alpha):
    B, S, D = x.shape
    F = W1.shape[1]
    E = eW1.shape[0]
    NBR = 3

    FBLK = 512 if F % 512 == 0 else F
    NFB = F // FBLK

    wr_pad = jnp.zeros((D, _LANES), jnp.float32).at[:, :E].set(Wr)
    alpha_arr = jnp.reshape(alpha.astype(jnp.float32), (1,))
    idx_pad, gate_pad = pl.pallas_call(
        functools.partial(_router_body, E, S),
        grid=(B,),
        in_specs=[
            pl.BlockSpec(memory_space=pltpu.SMEM),
            pl.BlockSpec((1, S, D), lambda b: (b, 0, 0)),
            pl.BlockSpec((D, _LANES), lambda b: (0, 0)),
        ],
        out_specs=[
            pl.BlockSpec((1, 1, _LANES), lambda b: (b, 0, 0)),
            pl.BlockSpec((1, 1, _LANES), lambda b: (b, 0, 0)),
        ],
        out_shape=[
            jax.ShapeDtypeStruct((B, 1, _LANES), jnp.int32),
            jax.ShapeDtypeStruct((B, 1, _LANES), jnp.float32),
        ],
        compiler_params=pltpu.CompilerParams(
            vmem_limit_bytes=100 * 1024 * 1024,
        ),
    )(alpha_arr, x, wr_pad)
    bidx = idx_pad[:, 0, :NBR]
    gates = gate_pad[:, 0, :NBR]

    grid_spec = pltpu.PrefetchScalarGridSpec(
        num_scalar_prefetch=2,
        grid=(B, NFB, NBR),
        in_specs=[
            pl.BlockSpec((1, S, D), lambda b, fb, br, bidx, gates: (b, 0, 0)),
            pl.BlockSpec((D, FBLK), lambda b, fb, br, bidx, gates: (0, fb)),
            pl.BlockSpec((1, FBLK), lambda b, fb, br, bidx, gates: (0, fb)),
            pl.BlockSpec((FBLK, D), lambda b, fb, br, bidx, gates: (fb, 0)),
            pl.BlockSpec((1, D), lambda b, fb, br, bidx, gates: (0, 0)),
            pl.BlockSpec((1, D, FBLK),
                         lambda b, fb, br, bidx, gates: (bidx[b, br], 0, fb)),
            pl.BlockSpec((1, 1, FBLK),
                         lambda b, fb, br, bidx, gates: (bidx[b, br], 0, fb)),
            pl.BlockSpec((1, FBLK, D),
                         lambda b, fb, br, bidx, gates: (bidx[b, br], fb, 0)),
            pl.BlockSpec((1, 1, D),
                         lambda b, fb, br, bidx, gates: (bidx[b, br], 0, 0)),
        ],
        out_specs=pl.BlockSpec((1, S, D),
                               lambda b, fb, br, bidx, gates: (b, 0, 0)),
    )
    out = pl.pallas_call(
        _dispatch_body,
        grid_spec=grid_spec,
        out_shape=jax.ShapeDtypeStruct((B, S, D), jnp.float32),
        compiler_params=pltpu.CompilerParams(
            dimension_semantics=("arbitrary", "arbitrary", "arbitrary"),
            vmem_limit_bytes=100 * 1024 * 1024,
        ),
    )(bidx, gates, x, W1, b1[None, :], W2, b2[None, :],
      eW1, eb1[:, None, :], eW2, eb2[:, None, :])
    return out
